# baseline (speedup 1.0000x reference)
import jax
import jax.numpy as jnp
from jax import lax
from jax.experimental import pallas as pl
from jax.experimental.pallas import tpu as pltpu, tpu_sc as plsc

B = 1024
L = 200
V = 100001
TOK = B * L
NC = 2
NS = 16
NW = NC * NS
UPW = B // NW
TPW = TOK // NW
G = 128
NG = TPW // G
NEVAL = 60000


def _full(c):
    return jnp.full((16,), c, jnp.int32)


def _body(log_ref, t1_ref, t2_ref, user_ref, mt_ref, wt_ref, we_ref,
          out_ref, itemv, t1v, t2v, midx, widx, ridx, uloc, rbuf,
          mbuf, wbuf, otile, m_sem, w_sem, r_sem):
    wid = lax.axis_index("s") * NC + lax.axis_index("c")
    iota = lax.iota(jnp.int32, 16)

    pltpu.sync_copy(user_ref.at[pl.ds(wid * UPW, UPW)], uloc)
    for i in range(2):
        u = uloc[pl.ds(i * 16, 16)]
        r = (u - 1) * 6
        r = jnp.where(r < 0, r + NEVAL, r)
        for k in range(6):
            plsc.store_scatter(ridx, [iota * 6 + (i * 96 + k)], r + k)
    c0 = pltpu.async_copy(we_ref.at[ridx.at[pl.ds(0, 96)]],
                          rbuf.at[pl.ds(0, 96)], r_sem)
    c1 = pltpu.async_copy(we_ref.at[ridx.at[pl.ds(96, 96)]],
                          rbuf.at[pl.ds(96, 96)], r_sem)
    c0.wait()
    c1.wait()

    def group(g, _):
        t0 = wid * TPW + g * G
        pltpu.sync_copy(log_ref.at[pl.ds(t0, G)], itemv)
        pltpu.sync_copy(t1_ref.at[pl.ds(t0, G)], t1v)
        pltpu.sync_copy(t2_ref.at[pl.ds(t0, G)], t2v)
        for j in range(8):
            it = itemv[pl.ds(j * 16, 16)]
            midx[pl.ds(j * 16, 16)] = it * 12 + t1v[pl.ds(j * 16, 16)]
            widx[pl.ds(j * 16, 16)] = it * 53 + t2v[pl.ds(j * 16, 16)]
        cm = pltpu.async_copy(mt_ref.at[midx], mbuf, m_sem)
        cw = pltpu.async_copy(wt_ref.at[widx], wbuf, w_sem)
        for j in range(8):
            rows = j * 16 + iota
            t = t0 + rows
            bb = t // 200
            ll = t - bb * 200
            rowb = (bb - wid * UPW) * 6
            for k in range(6):
                v = plsc.load_gather(rbuf, [rowb + k, ll])
                plsc.store_scatter(otile, [rows, _full(18 + k)], v)
        cm.wait()
        cw.wait()
        for j in range(8):
            rows = j * 16 + iota
            for k in range(12):
                v = plsc.load_gather(mbuf, [rows, _full(k)])
                plsc.store_scatter(otile, [rows, _full(k)], v)
            for k in range(6):
                v = plsc.load_gather(wbuf, [rows, _full(k)])
                plsc.store_scatter(otile, [rows, _full(12 + k)], v)
        pltpu.sync_copy(otile, out_ref.at[pl.ds(t0, G)])
        return 0

    lax.fori_loop(0, NG, group, 0)


@jax.jit
def _run(log_flat, t1_flat, t2_flat, user, mt, wt, we):
    mesh = plsc.VectorSubcoreMesh(core_axis_name="c", subcore_axis_name="s")
    f = pl.kernel(
        _body,
        out_type=jax.ShapeDtypeStruct((TOK, 24), jnp.float32),
        mesh=mesh,
        compiler_params=pltpu.CompilerParams(
            needs_layout_passes=False, use_tc_tiling_on_sc=False),
        scratch_types=[
            pltpu.VMEM((G,), jnp.int32),
            pltpu.VMEM((G,), jnp.int32),
            pltpu.VMEM((G,), jnp.int32),
            pltpu.VMEM((G,), jnp.int32),
            pltpu.VMEM((G,), jnp.int32),
            pltpu.VMEM((192,), jnp.int32),
            pltpu.VMEM((UPW,), jnp.int32),
            pltpu.VMEM((UPW * 6, L), jnp.float32),
            pltpu.VMEM((G, 16), jnp.float32),
            pltpu.VMEM((G, 8), jnp.float32),
            pltpu.VMEM((G, 24), jnp.float32),
            pltpu.SemaphoreType.DMA,
            pltpu.SemaphoreType.DMA,
            pltpu.SemaphoreType.DMA,
        ],
    )
    return f(log_flat, t1_flat, t2_flat, user, mt, wt, we)


def kernel(log_seqs, time1_seqs, time2_seqs, user, month_pop_table,
           week_pop_table, week_eval_pop):
    mt = jnp.pad(month_pop_table.reshape(12, 12, V).transpose(2, 0, 1),
                 ((0, 0), (0, 0), (0, 4))).reshape(12 * V, 16)
    wt = jnp.pad(week_pop_table.reshape(53, 6, V).transpose(2, 0, 1),
                 ((0, 0), (0, 0), (0, 2))).reshape(53 * V, 8)
    out = _run(
        log_seqs.reshape(-1).astype(jnp.int32),
        time1_seqs.reshape(-1).astype(jnp.int32),
        time2_seqs.reshape(-1).astype(jnp.int32),
        user.astype(jnp.int32),
        mt, wt, week_eval_pop)
    return lax.stop_gradient(out.reshape(B, L, 24))


# speedup vs baseline: 5.3704x; 5.3704x over previous
import jax
import jax.numpy as jnp
from jax import lax
from jax.experimental import pallas as pl
from jax.experimental.pallas import tpu as pltpu, tpu_sc as plsc

B = 1024
L = 200
V = 100001
TOK = B * L
NC = 2
NS = 16
NW = NC * NS
UPW = B // NW
TPW = TOK // NW
G = 128
NG = TPW // G
NEVAL = 60000


def _full(c):
    return jnp.full((16,), c, jnp.int32)


def _body(log_ref, t1_ref, t2_ref, user_ref, mt_ref, wt_ref, we_ref,
          out_ref, itemv, t1v, t2v, midx, widx, ridx, uloc, rbuf,
          mkbuf, wkbuf, otile, m_sem, w_sem, r_sem):
    wid = lax.axis_index("s") * NC + lax.axis_index("c")
    iota = lax.iota(jnp.int32, 16)

    pltpu.sync_copy(user_ref.at[pl.ds(wid * UPW, UPW)], uloc)
    for i in range(2):
        u = uloc[pl.ds(i * 16, 16)]
        r = (u - 1) * 6
        r = jnp.where(r < 0, r + NEVAL, r)
        for k in range(6):
            plsc.store_scatter(ridx, [iota * 6 + (i * 96 + k)], r + k)
    c0 = pltpu.async_copy(we_ref.at[ridx.at[pl.ds(0, 96)]],
                          rbuf.at[pl.ds(0, 96)], r_sem)
    c1 = pltpu.async_copy(we_ref.at[ridx.at[pl.ds(96, 96)]],
                          rbuf.at[pl.ds(96, 96)], r_sem)
    c0.wait()
    c1.wait()

    def group(g, _):
        t0 = wid * TPW + g * G
        pltpu.sync_copy(log_ref.at[pl.ds(t0, G)], itemv)
        pltpu.sync_copy(t1_ref.at[pl.ds(t0, G)], t1v)
        pltpu.sync_copy(t2_ref.at[pl.ds(t0, G)], t2v)
        for j in range(8):
            it = itemv[pl.ds(j * 16, 16)]
            mb = t1v[pl.ds(j * 16, 16)] * (12 * V) + it
            wb = t2v[pl.ds(j * 16, 16)] * (6 * V) + it
            for k in range(12):
                midx[pl.ds(k * G + j * 16, 16)] = mb + k * V
            for k in range(6):
                widx[pl.ds(k * G + j * 16, 16)] = wb + k * V
        cs = []
        for k in range(12):
            cs.append(pltpu.async_copy(
                mt_ref.at[midx.at[pl.ds(k * G, G)]], mkbuf.at[k], m_sem))
        for k in range(6):
            cs.append(pltpu.async_copy(
                wt_ref.at[widx.at[pl.ds(k * G, G)]], wkbuf.at[k], w_sem))
        for j in range(8):
            rows = j * 16 + iota
            t = t0 + rows
            bb = t // 200
            ll = t - bb * 200
            rowb = (bb - wid * UPW) * 6
            for k in range(6):
                v = plsc.load_gather(rbuf, [rowb + k, ll])
                plsc.store_scatter(otile, [rows, _full(18 + k)], v)
        for c in cs:
            c.wait()
        for j in range(8):
            rows = j * 16 + iota
            for k in range(12):
                plsc.store_scatter(otile, [rows, _full(k)],
                                   mkbuf[k, pl.ds(j * 16, 16)])
            for k in range(6):
                plsc.store_scatter(otile, [rows, _full(12 + k)],
                                   wkbuf[k, pl.ds(j * 16, 16)])
        pltpu.sync_copy(otile, out_ref.at[pl.ds(t0, G)])
        return 0

    lax.fori_loop(0, NG, group, 0)


@jax.jit
def _run(log_flat, t1_flat, t2_flat, user, mt_flat, wt_flat, we):
    mesh = plsc.VectorSubcoreMesh(core_axis_name="c", subcore_axis_name="s")
    f = pl.kernel(
        _body,
        out_type=jax.ShapeDtypeStruct((TOK, 24), jnp.float32),
        mesh=mesh,
        compiler_params=pltpu.CompilerParams(
            needs_layout_passes=False, use_tc_tiling_on_sc=False),
        scratch_types=[
            pltpu.VMEM((G,), jnp.int32),
            pltpu.VMEM((G,), jnp.int32),
            pltpu.VMEM((G,), jnp.int32),
            pltpu.VMEM((12 * G,), jnp.int32),
            pltpu.VMEM((6 * G,), jnp.int32),
            pltpu.VMEM((192,), jnp.int32),
            pltpu.VMEM((UPW,), jnp.int32),
            pltpu.VMEM((UPW * 6, L), jnp.float32),
            pltpu.VMEM((12, G), jnp.float32),
            pltpu.VMEM((6, G), jnp.float32),
            pltpu.VMEM((G, 24), jnp.float32),
            pltpu.SemaphoreType.DMA,
            pltpu.SemaphoreType.DMA,
            pltpu.SemaphoreType.DMA,
        ],
    )
    return f(log_flat, t1_flat, t2_flat, user, mt_flat, wt_flat, we)


def kernel(log_seqs, time1_seqs, time2_seqs, user, month_pop_table,
           week_pop_table, week_eval_pop):
    out = _run(
        log_seqs.reshape(-1).astype(jnp.int32),
        time1_seqs.reshape(-1).astype(jnp.int32),
        time2_seqs.reshape(-1).astype(jnp.int32),
        user.astype(jnp.int32),
        month_pop_table.reshape(-1),
        week_pop_table.reshape(-1),
        week_eval_pop)
    return lax.stop_gradient(out.reshape(B, L, 24))
